# baseline (speedup 1.0000x reference)
import functools

import jax
import jax.numpy as jnp
from jax import lax
from jax.experimental import pallas as pl
from jax.experimental.pallas import tpu as pltpu
from jax.experimental.pallas import tpu_sc as plsc

DIM = 64
LANE = 128
NC, NS = 2, 16
NW = NC * NS

_PARAMS = pltpu.CompilerParams(use_tc_tiling_on_sc=False, needs_layout_passes=False)


_PARAMS_TILED = pltpu.CompilerParams(
    use_tc_tiling_on_sc=True, needs_layout_passes=False
)


@functools.cache
def _phase_a(V):
    VT = V // LANE
    NJ = VT // NW
    JTM = NJ * NW
    EXTRA = VT - JTM
    TAIL = (V - VT * LANE) // 2
    NG = NJ // 2
    mesh = plsc.VectorSubcoreMesh(core_axis_name="c", subcore_axis_name="s")

    @functools.partial(
        pl.kernel,
        mesh=mesh,
        out_type=jax.ShapeDtypeStruct((V // 2, 2 * LANE // 2), jnp.float32),
        scratch_types=[
            pltpu.VMEM((DIM, LANE), jnp.float32),
            pltpu.VMEM((DIM, LANE), jnp.float32),
            pltpu.VMEM((DIM, LANE), jnp.float32),
            pltpu.VMEM((DIM, LANE), jnp.float32),
            pltpu.SemaphoreType.DMA,
            pltpu.SemaphoreType.DMA,
            pltpu.SemaphoreType.DMA,
            pltpu.SemaphoreType.DMA,
        ],
        compiler_params=_PARAMS_TILED,
    )
    def k(et_hbm, tail_hbm, wp_hbm, n0, n1, r0, r1, ns0, ns1, rs0, rs1):
        w = lax.axis_index("s") * NC + lax.axis_index("c")
        nb = (n0, n1)
        rb = (r0, r1)
        nsem = (ns0, ns1)
        rsem = (rs0, rs1)
        iota = lax.iota(jnp.int32, 16)
        io8 = iota // 8
        ior7 = iota & 7

        def stage(j, nref, sem):
            for ib in range(8):
                pltpu.async_copy(
                    et_hbm.at[pl.ds(8 * ib, 8), pl.ds(LANE * j, LANE)],
                    nref.at[pl.ds(8 * ib, 8)],
                    sem,
                )

        def wait_stage(j, nref, sem):
            for ib in range(8):
                pltpu.make_async_copy(
                    et_hbm.at[pl.ds(8 * ib, 8), pl.ds(LANE * j, LANE)],
                    nref.at[pl.ds(8 * ib, 8)],
                    sem,
                ).wait()

        def transpose_block(n_ref, r_ref, nct):
            def dgrp(dg, carry):
                dvec = iota + 16 * dg
                for j in range(16):
                    m = (iota + j) & 15
                    mh = m >> 1
                    col = ((m & 1) << 6) + dvec
                    for ct in range(nct):
                        vec = plsc.load_gather(n_ref, [dvec, m + 16 * ct])
                        plsc.store_scatter(r_ref, [mh + 8 * ct, col], vec)
                return carry

            lax.fori_loop(0, 4, dgrp, 0)

        stage(w, n0, ns0)

        def tile(g, par):
            jj = 2 * g + par
            j = w + NW * jj
            wait_stage(j, nb[par], nsem[par])

            def issue_next():
                stage(j + NW, nb[1 - par], nsem[1 - par])

            if par == 0:
                issue_next()
            else:
                pl.when(g < NG - 1)(issue_next)

            def drain_r():
                jprev = w + NW * (jj - 2)
                pltpu.make_async_copy(
                    rb[par], wp_hbm.at[pl.ds(DIM * jprev, DIM)], rsem[par]
                ).wait()

            pl.when(g >= 1)(drain_r)
            transpose_block(nb[par], rb[par], 8)
            pltpu.async_copy(
                rb[par], wp_hbm.at[pl.ds(DIM * j, DIM)], rsem[par]
            )

        def gbody(g, carry):
            tile(g, 0)
            tile(g, 1)
            return carry

        lax.fori_loop(0, NG, gbody, 0)
        for par in range(2):
            jlast = w + NW * (NJ - 2 + par)
            pltpu.make_async_copy(
                rb[par], wp_hbm.at[pl.ds(DIM * jlast, DIM)], rsem[par]
            ).wait()

        @pl.when(w < EXTRA)
        def _extra():
            j = JTM + w
            stage(j, n0, ns0)
            wait_stage(j, n0, ns0)
            transpose_block(n0, r0, 8)
            pltpu.sync_copy(r0, wp_hbm.at[pl.ds(DIM * j, DIM)])

        if TAIL:
            @pl.when(w == EXTRA)
            def _tail():
                pltpu.sync_copy(tail_hbm, wp_hbm.at[pl.ds(DIM * VT, TAIL)])

    return k


@functools.cache
def _phase_b(V, B, T):
    JT = B // LANE
    assert JT == NW
    IB = T // 8
    mesh = plsc.VectorSubcoreMesh(core_axis_name="c", subcore_axis_name="s")

    @functools.partial(
        pl.kernel,
        mesh=mesh,
        out_type=jax.ShapeDtypeStruct((T, DIM // 8, JT, 8, LANE), jnp.float32),
        scratch_types=[
            pltpu.VMEM((IB, 8, LANE), jnp.int32),
            pltpu.VMEM((LANE, DIM), jnp.float32),
            pltpu.VMEM((LANE, DIM), jnp.float32),
            pltpu.VMEM((DIM // 8, 8, LANE), jnp.float32),
            pltpu.VMEM((DIM // 8, 8, LANE), jnp.float32),
            pltpu.SemaphoreType.DMA,
            pltpu.SemaphoreType.DMA,
            pltpu.SemaphoreType.DMA,
            pltpu.SemaphoreType.DMA,
        ],
        compiler_params=_PARAMS,
    )
    def k(xl_hbm, wrm_hbm, out_hbm, idx_v, g0, g1, t0, t1, gs0, gs1, ts0, ts1):
        w = lax.axis_index("s") * NC + lax.axis_index("c")
        gb = (g0, g1)
        tb = (t0, t1)
        gsem = (gs0, gs1)
        tsem = (ts0, ts1)
        pltpu.sync_copy(xl_hbm.at[:, w], idx_v)
        iota = lax.iota(jnp.int32, 16)
        zero16 = iota * 0
        c_vecs = [iota + 16 * c0 for c0 in range(8)]

        pltpu.async_copy(wrm_hbm.at[idx_v.at[0, 0]], g0, gs0)

        def chunk(g, par):
            t = 2 * g + par
            i = t // 8
            r = lax.rem(t, 8)
            pltpu.make_async_copy(
                wrm_hbm.at[idx_v.at[i, r]], gb[par], gsem[par]
            ).wait()

            t1n = t + 1
            i1 = t1n // 8
            r1 = lax.rem(t1n, 8)

            def issue_next():
                pltpu.async_copy(
                    wrm_hbm.at[idx_v.at[i1, r1]], gb[1 - par], gsem[1 - par]
                )

            if par == 0:
                issue_next()
            else:
                pl.when(g < T // 2 - 1)(issue_next)

            def drain_t():
                tprev = t - 2
                pltpu.make_async_copy(
                    tb[par], out_hbm.at[tprev, :, w], tsem[par]
                ).wait()

            pl.when(g >= 1)(drain_t)

            def dgrp(dg, carry):
                for j in range(16):
                    m = (iota + j) & 15
                    d_idx = m + 16 * dg
                    ib_v = (m >> 3) + 2 * dg
                    r_v = m & 7
                    for c0 in range(8):
                        vec = plsc.load_gather(gb[par], [c_vecs[c0], d_idx])
                        plsc.store_scatter(
                            tb[par], [ib_v, r_v, c_vecs[c0]], vec
                        )
                return carry

            lax.fori_loop(0, 4, dgrp, 0)
            pltpu.async_copy(tb[par], out_hbm.at[t, :, w], tsem[par])

        def gbody(g, carry):
            chunk(g, 0)
            chunk(g, 1)
            return carry

        lax.fori_loop(0, T // 2, gbody, 0)
        for par in range(2):
            pltpu.make_async_copy(
                tb[par], out_hbm.at[T - 2 + par, :, w], tsem[par]
            ).wait()

    return k


def kernel(x, emb_weight):
    B, T = x.shape
    V = emb_weight.shape[0]
    xl = (
        x.astype(jnp.int32)
        .T.reshape(T // 8, 8, B // LANE, LANE)
        .transpose(0, 2, 1, 3)
    )
    VT = V // LANE
    tail2 = emb_weight[VT * LANE :].reshape((V - VT * LANE) // 2, LANE)
    wp = _phase_a(V)(emb_weight.T, tail2)
    out5 = _phase_b(V, B, T)(xl, wp.reshape(V, DIM))
    return out5.transpose(2, 4, 0, 1, 3).reshape(B, T, DIM)


# speedup vs baseline: 2.1681x; 2.1681x over previous
import functools

import jax
import jax.numpy as jnp
from jax import lax
from jax.experimental import pallas as pl
from jax.experimental.pallas import tpu as pltpu
from jax.experimental.pallas import tpu_sc as plsc

DIM = 64
LANE = 128
NC, NS = 2, 16
NW = NC * NS

_PARAMS = pltpu.CompilerParams(use_tc_tiling_on_sc=False, needs_layout_passes=False)


_PARAMS_TILED = pltpu.CompilerParams(
    use_tc_tiling_on_sc=True, needs_layout_passes=False
)


@functools.cache
def _phase_a(V):
    VT = V // LANE
    NJ = VT // NW
    JTM = NJ * NW
    EXTRA = VT - JTM
    TAIL = (V - VT * LANE) // 2
    NG = NJ // 2
    mesh = plsc.VectorSubcoreMesh(core_axis_name="c", subcore_axis_name="s")

    @functools.partial(
        pl.kernel,
        mesh=mesh,
        out_type=jax.ShapeDtypeStruct((V // 2, 2 * LANE // 2), jnp.float32),
        scratch_types=[
            pltpu.VMEM((DIM // 8, 8, LANE), jnp.float32),
            pltpu.VMEM((DIM // 8, 8, LANE), jnp.float32),
            pltpu.VMEM((DIM, LANE), jnp.float32),
            pltpu.VMEM((DIM, LANE), jnp.float32),
            pltpu.SemaphoreType.DMA,
            pltpu.SemaphoreType.DMA,
            pltpu.SemaphoreType.DMA,
            pltpu.SemaphoreType.DMA,
        ],
        compiler_params=_PARAMS_TILED,
    )
    def k(et_hbm, tail_hbm, wp_hbm, n0, n1, r0, r1, ns0, ns1, rs0, rs1):
        w = lax.axis_index("s") * NC + lax.axis_index("c")
        nb = (n0, n1)
        rb = (r0, r1)
        nsem = (ns0, ns1)
        rsem = (rs0, rs1)
        iota = lax.iota(jnp.int32, 16)
        io8 = iota // 8
        ior7 = iota & 7

        def stage(j, nref, sem):
            for ib in range(8):
                pltpu.async_copy(
                    et_hbm.at[pl.ds(8 * ib, 8), pl.ds(LANE * j, LANE)],
                    nref.at[ib],
                    sem,
                )

        def wait_stage(j, nref, sem):
            for ib in range(8):
                pltpu.make_async_copy(
                    et_hbm.at[pl.ds(8 * ib, 8), pl.ds(LANE * j, LANE)],
                    nref.at[ib],
                    sem,
                ).wait()

        def transpose_block(n_ref, r_ref, nct):
            def dgrp(dg, carry):
                ib_l = io8 + 2 * dg
                dvec = iota + 16 * dg
                for j in range(16):
                    m = (iota + j) & 15
                    mh = m >> 1
                    col = ((m & 1) << 6) + dvec
                    vecs = [
                        plsc.load_gather(n_ref, [ib_l, ior7, m + 16 * ct])
                        for ct in range(nct)
                    ]
                    for ct in range(nct):
                        plsc.store_scatter(r_ref, [mh + 8 * ct, col], vecs[ct])
                return carry

            lax.fori_loop(0, 4, dgrp, 0)

        stage(w, n0, ns0)

        def tile(g, par):
            jj = 2 * g + par
            j = w + NW * jj
            wait_stage(j, nb[par], nsem[par])

            def issue_next():
                stage(j + NW, nb[1 - par], nsem[1 - par])

            if par == 0:
                issue_next()
            else:
                pl.when(g < NG - 1)(issue_next)

            def drain_r():
                jprev = w + NW * (jj - 2)
                pltpu.make_async_copy(
                    rb[par], wp_hbm.at[pl.ds(DIM * jprev, DIM)], rsem[par]
                ).wait()

            pl.when(g >= 1)(drain_r)
            transpose_block(nb[par], rb[par], 8)
            pltpu.async_copy(
                rb[par], wp_hbm.at[pl.ds(DIM * j, DIM)], rsem[par]
            )

        def gbody(g, carry):
            tile(g, 0)
            tile(g, 1)
            return carry

        lax.fori_loop(0, NG, gbody, 0)
        for par in range(2):
            jlast = w + NW * (NJ - 2 + par)
            pltpu.make_async_copy(
                rb[par], wp_hbm.at[pl.ds(DIM * jlast, DIM)], rsem[par]
            ).wait()

        @pl.when(w < EXTRA)
        def _extra():
            j = JTM + w
            stage(j, n0, ns0)
            wait_stage(j, n0, ns0)
            transpose_block(n0, r0, 8)
            pltpu.sync_copy(r0, wp_hbm.at[pl.ds(DIM * j, DIM)])

        if TAIL:
            @pl.when(w == EXTRA)
            def _tail():
                pltpu.sync_copy(tail_hbm, wp_hbm.at[pl.ds(DIM * VT, TAIL)])

    return k


@functools.cache
def _phase_b(V, B, T):
    JT = B // LANE
    assert JT == NW
    IB = T // 8
    mesh = plsc.VectorSubcoreMesh(core_axis_name="c", subcore_axis_name="s")

    @functools.partial(
        pl.kernel,
        mesh=mesh,
        out_type=jax.ShapeDtypeStruct((T, DIM // 8, JT, 8, LANE), jnp.float32),
        scratch_types=[
            pltpu.VMEM((IB, 8, LANE), jnp.int32),
            pltpu.VMEM((LANE, DIM), jnp.float32),
            pltpu.VMEM((LANE, DIM), jnp.float32),
            pltpu.VMEM((DIM // 8, 8, LANE), jnp.float32),
            pltpu.VMEM((DIM // 8, 8, LANE), jnp.float32),
            pltpu.SemaphoreType.DMA,
            pltpu.SemaphoreType.DMA,
            pltpu.SemaphoreType.DMA,
            pltpu.SemaphoreType.DMA,
        ],
        compiler_params=_PARAMS,
    )
    def k(xl_hbm, wrm_hbm, out_hbm, idx_v, g0, g1, t0, t1, gs0, gs1, ts0, ts1):
        w = lax.axis_index("s") * NC + lax.axis_index("c")
        gb = (g0, g1)
        tb = (t0, t1)
        gsem = (gs0, gs1)
        tsem = (ts0, ts1)
        pltpu.sync_copy(xl_hbm.at[:, w], idx_v)
        iota = lax.iota(jnp.int32, 16)
        zero16 = iota * 0
        c_vecs = [iota + 16 * c0 for c0 in range(8)]

        pltpu.async_copy(wrm_hbm.at[idx_v.at[0, 0]], g0, gs0)

        def chunk(g, par):
            t = 2 * g + par
            i = t // 8
            r = lax.rem(t, 8)
            pltpu.make_async_copy(
                wrm_hbm.at[idx_v.at[i, r]], gb[par], gsem[par]
            ).wait()

            t1n = t + 1
            i1 = t1n // 8
            r1 = lax.rem(t1n, 8)

            def issue_next():
                pltpu.async_copy(
                    wrm_hbm.at[idx_v.at[i1, r1]], gb[1 - par], gsem[1 - par]
                )

            if par == 0:
                issue_next()
            else:
                pl.when(g < T // 2 - 1)(issue_next)

            def drain_t():
                tprev = t - 2
                pltpu.make_async_copy(
                    tb[par], out_hbm.at[tprev, :, w], tsem[par]
                ).wait()

            pl.when(g >= 1)(drain_t)

            def dgrp(dg, carry):
                for j in range(16):
                    m = (iota + j) & 15
                    d_idx = m + 16 * dg
                    ib_v = (m >> 3) + 2 * dg
                    r_v = m & 7
                    vecs = [
                        plsc.load_gather(gb[par], [c_vecs[c0], d_idx])
                        for c0 in range(8)
                    ]
                    for c0 in range(8):
                        plsc.store_scatter(
                            tb[par], [ib_v, r_v, c_vecs[c0]], vecs[c0]
                        )
                return carry

            lax.fori_loop(0, 4, dgrp, 0)
            pltpu.async_copy(tb[par], out_hbm.at[t, :, w], tsem[par])

        def gbody(g, carry):
            chunk(g, 0)
            chunk(g, 1)
            return carry

        lax.fori_loop(0, T // 2, gbody, 0)
        for par in range(2):
            pltpu.make_async_copy(
                tb[par], out_hbm.at[T - 2 + par, :, w], tsem[par]
            ).wait()

    return k


def kernel(x, emb_weight):
    B, T = x.shape
    V = emb_weight.shape[0]
    xl = (
        x.astype(jnp.int32)
        .T.reshape(T // 8, 8, B // LANE, LANE)
        .transpose(0, 2, 1, 3)
    )
    VT = V // LANE
    tail2 = emb_weight[VT * LANE :].reshape((V - VT * LANE) // 2, LANE)
    wp = _phase_a(V)(emb_weight.T, tail2)
    out5 = _phase_b(V, B, T)(xl, wp.reshape(V, DIM))
    return out5.transpose(2, 4, 0, 1, 3).reshape(B, T, DIM)
